# baseline (speedup 1.0000x reference)
import functools

import jax
import jax.numpy as jnp
from jax import lax
from jax.experimental import pallas as pl
from jax.experimental.pallas import tpu as pltpu
from jax.experimental.pallas import tpu_sc as plsc

N_T = 50000
N_F = 50000
E = 800000
D_T = 64
D_F = 64
D_SKIP = 32

BLK = 1000


def _pre_body(nodes_ref, q_in_ref, t0_ref,
              kW0, kb0, kW1, kb1, kW2, kb2,
              vW0, vb0, vW1, vb1, vW2, vb2,
              qW0, qb0, qW1, qb1, qW2, qb2,
              ctab_ref, qtab_ref):
    x = nodes_ref[...]
    h = jax.nn.relu(jnp.dot(x, kW0[...].T, preferred_element_type=jnp.float32) + kb0[...])
    h = jax.nn.relu(jnp.dot(h, kW1[...].T, preferred_element_type=jnp.float32) + kb1[...])
    k = jnp.dot(h, kW2[...].T, preferred_element_type=jnp.float32) + kb2[...]
    h = jax.nn.relu(jnp.dot(x, vW0[...].T, preferred_element_type=jnp.float32) + vb0[...])
    h = jax.nn.relu(jnp.dot(h, vW1[...].T, preferred_element_type=jnp.float32) + vb1[...])
    v = jnp.dot(h, vW2[...].T, preferred_element_type=jnp.float32) + vb2[...]
    xq = q_in_ref[...]
    h = jax.nn.relu(jnp.dot(xq, qW0[...].T, preferred_element_type=jnp.float32) + qb0[...])
    h = jax.nn.relu(jnp.dot(h, qW1[...].T, preferred_element_type=jnp.float32) + qb1[...])
    q = jnp.dot(h, qW2[...].T, preferred_element_type=jnp.float32) + qb2[...]
    z2 = jnp.zeros((x.shape[0], 2), dtype=jnp.float32)
    ctab_ref[...] = jnp.concatenate([k, z2, v, t0_ref[...]], axis=1)
    norm = jnp.float32(1.0) / jnp.sqrt(jnp.float32(30.0))
    z98 = jnp.zeros((x.shape[0], 98), dtype=jnp.float32)
    qtab_ref[...] = jnp.concatenate([q * norm, z98], axis=1)


def _whole(arr2d):
    return pl.BlockSpec(arr2d, lambda i: (0, 0))


def _pre_stage(nodes_in, q_in, t0, ws):
    row = pl.BlockSpec((BLK, None), lambda i: (i, 0))

    def wspec(a):
        return pl.BlockSpec((a.shape[0], a.shape[1]), lambda i: (0, 0))

    def bspec(a):
        return pl.BlockSpec((1, a.shape[1]), lambda i: (0, 0))

    weights = [ws['kW0'], ws['kb0'], ws['kW1'], ws['kb1'], ws['kW2'], ws['kb2'],
               ws['vW0'], ws['vb0'], ws['vW1'], ws['vb1'], ws['vW2'], ws['vb2'],
               ws['qW0'], ws['qb0'], ws['qW1'], ws['qb1'], ws['qW2'], ws['qb2']]
    in_specs = [pl.BlockSpec((BLK, nodes_in.shape[1]), lambda i: (i, 0)),
                pl.BlockSpec((BLK, q_in.shape[1]), lambda i: (i, 0)),
                pl.BlockSpec((BLK, t0.shape[1]), lambda i: (i, 0))]
    for w in weights:
        if w.ndim == 1:
            in_specs.append(bspec(w.reshape(1, -1)))
        else:
            in_specs.append(wspec(w))
    weights_r = [w.reshape(1, -1) if w.ndim == 1 else w for w in weights]
    grid = N_T // BLK
    ctab, qtab = pl.pallas_call(
        _pre_body,
        grid=(grid,),
        in_specs=in_specs,
        out_specs=[pl.BlockSpec((BLK, 128), lambda i: (i, 0)),
                   pl.BlockSpec((BLK, 128), lambda i: (i, 0))],
        out_shape=[jax.ShapeDtypeStruct((N_T, 128), jnp.float32),
                   jax.ShapeDtypeStruct((N_F, 128), jnp.float32)],
    )(nodes_in, q_in, t0, *weights_r)
    return ctab, qtab


def _post_body(pa_ref, pb_ref, fs_ref,
               gWih, gWhh, gbih, gbhh, ln_g, ln_b, mW0, mb0, mW1, mb1,
               out_ref):
    pa = pa_ref[...]
    pb = pb_ref[...]
    num = pa[:, :96] + pb[:, :96]
    den = pa[:, 96:97] + pb[:, 96:97]
    den = jnp.where(den == 0.0, 1.0, den)
    ws = num / den
    fs = fs_ref[...]
    gi = jnp.dot(ws, gWih[...].T, preferred_element_type=jnp.float32) + gbih[...]
    gh = jnp.dot(fs, gWhh[...].T, preferred_element_type=jnp.float32) + gbhh[...]
    i_r, i_z, i_n = gi[:, :64], gi[:, 64:128], gi[:, 128:]
    h_r, h_z, h_n = gh[:, :64], gh[:, 64:128], gh[:, 128:]
    r = jax.nn.sigmoid(i_r + h_r)
    z = jax.nn.sigmoid(i_z + h_z)
    nn = jnp.tanh(i_n + r * h_n)
    h = (1.0 - z) * nn + z * fs
    mu = jnp.mean(h, axis=1, keepdims=True)
    var = jnp.mean((h - mu) ** 2, axis=1, keepdims=True)
    hn = (h - mu) * jax.lax.rsqrt(var + 1e-05) * ln_g[...] + ln_b[...]
    o = jax.nn.relu(jnp.dot(hn, mW0[...].T, preferred_element_type=jnp.float32) + mb0[...])
    o = jnp.dot(o, mW1[...].T, preferred_element_type=jnp.float32) + mb1[...]
    out_ref[...] = fs + o


def _post_stage(pa, pb, fs, ws):
    def wspec(a):
        return pl.BlockSpec((a.shape[0], a.shape[1]), lambda i: (0, 0))

    weights = [ws['gWih'], ws['gWhh'], ws['gbih'], ws['gbhh'], ws['ln_g'],
               ws['ln_b'], ws['mW0'], ws['mb0'], ws['mW1'], ws['mb1']]
    weights_r = [w.reshape(1, -1) if w.ndim == 1 else w for w in weights]
    in_specs = [pl.BlockSpec((BLK, pa.shape[1]), lambda i: (i, 0)),
                pl.BlockSpec((BLK, pb.shape[1]), lambda i: (i, 0)),
                pl.BlockSpec((BLK, fs.shape[1]), lambda i: (i, 0))]
    in_specs += [wspec(w) for w in weights_r]
    out = pl.pallas_call(
        _post_body,
        grid=(N_F // BLK,),
        in_specs=in_specs,
        out_specs=pl.BlockSpec((BLK, D_F), lambda i: (i, 0)),
        out_shape=jax.ShapeDtypeStruct((N_F, D_F), jnp.float32),
    )(pa, pb, fs, *weights_r)
    return out


NC = 2
NS = 16
L = 16
NW = NC * NS
EC = E // NW
EB = 64
NPASS = 14
QR = 3584
TPS = QR // NS
ROWW = 128
ACC_ROWS = QR + 8
DUMP = QR
ZR = 16
EPAD = 25088
NSCAN = EPAD // L
GPAD = 31 * EC + EPAD


def _edge_body(ctab, qtab, srcg, dstg, out,
               srcv, dstv, comb0, comb1, qb0, qb1,
               wst0, wst1, idx0, idx1, zbuf,
               accum, semg0, semg1, sems0, sems1):
    c = lax.axis_index("c")
    s = lax.axis_index("s")
    w = s * NC + c
    base_w = w * EC
    iv = lax.iota(jnp.int32, L)
    zv = jnp.zeros((L,), jnp.float32)

    combs = (comb0, comb1)
    qbufs = (qb0, qb1)
    wsts = (wst0, wst1)
    idxs = (idx0, idx1)
    semgs = (semg0, semg1)
    semss = (sems0, sems1)

    def zb_body(r, _):
        rr = jnp.broadcast_to(r, (L,)).astype(jnp.int32)
        for j in range(ROWW // L):
            plsc.store_scatter(zbuf, [rr, iv + j * L], zv)
        return 0
    lax.fori_loop(0, ZR, zb_body, 0)

    with jax.named_scope("idx_preload"):
        pltpu.sync_copy(srcg.at[pl.ds(base_w, EPAD)], srcv)
        pltpu.sync_copy(dstg.at[pl.ds(base_w, EPAD)], dstv)

    def cbody(g, cs):
        d = dstv[pl.ds(g * L, L)]
        gidx = g * L + iv
        d = jnp.where(gidx < EC, d, jnp.int32(2 ** 30))
        return tuple(
            cs[t] + jnp.sum((d < (t + 1) * QR).astype(jnp.int32))
            for t in range(NPASS - 1))
    with jax.named_scope("count_scan"):
        cs = lax.fori_loop(0, NSCAN, cbody,
                           tuple(jnp.int32(0) for _ in range(NPASS - 1)))

    def issue(bstart, par):
        pltpu.async_copy(ctab.at[srcv.at[pl.ds(bstart, EB)]],
                         combs[par], semgs[par])
        pltpu.async_copy(qtab.at[dstv.at[pl.ds(bstart, EB)]],
                         qbufs[par], semgs[par])

    def wait_gather(par):
        pltpu.make_async_copy(ctab.at[pl.ds(0, EB)], combs[par],
                              semgs[par]).wait()
        pltpu.make_async_copy(qtab.at[pl.ds(0, EB)], qbufs[par],
                              semgs[par]).wait()

    def wait_scatter(par):
        pltpu.make_async_copy(out.at[pl.ds(0, EB)],
                              accum.at[pl.ds(0, EB)], semss[par]).wait()

    def compute(bstart, par, lo, hi, pq):
        cb, qb, wb, ib = combs[par], qbufs[par], wsts[par], idxs[par]

        def gbody(g, _):
            rows16 = iv + g * L
            gidx = bstart + rows16
            d16 = dstv[pl.ds(bstart + g * L, L)]
            valid = (gidx >= lo) & (gidx < hi)
            dl = jnp.where(valid, d16 - pq, jnp.int32(DUMP))
            plsc.store_scatter(ib, [rows16], dl)
            att = jnp.zeros((L,), jnp.float32)
            for f in range(32):
                fc = jnp.full((L,), f, jnp.int32)
                att = att + (plsc.load_gather(cb, [rows16, fc]) *
                             plsc.load_gather(qb, [rows16, fc]))
            ex16 = jnp.exp(att)
            plsc.store_scatter(wb, [rows16, jnp.full((L,), 96, jnp.int32)],
                               ex16)
            for e in range(L):
                row = g * L + e
                exe = ex16[e]
                for j in range(6):
                    vv = cb[row, pl.ds(32 + j * L, L)]
                    wb[row, pl.ds(j * L, L)] = exe * vv
            return 0
        lax.fori_loop(0, EB // L, gbody, 0)

    def pass_body(p, _):
        lo = jnp.int32(0)
        for t in range(NPASS - 1):
            lo = jnp.where(p == t + 1, cs[t], lo)
        hi = jnp.int32(EC)
        for t in range(NPASS - 1):
            hi = jnp.where(p == t, cs[t], hi)
        pq = p * QR
        def zr_body(i, _):
            pltpu.sync_copy(zbuf, accum.at[pl.ds(s * TPS + i * ZR, ZR)])
            return 0
        with jax.named_scope("zero"):
            lax.fori_loop(0, TPS // ZR, zr_body, 0)
            plsc.subcore_barrier()

        b0 = (lo // EB) * EB
        nb = (hi - b0 + (EB - 1)) // EB
        nbp = (nb + 1) // 2

        def pair_body(i2, _):
            for b in range(2):
                k = i2 * 2 + b

                @pl.when(k < nb)
                def _():
                    bstart = b0 + k * EB
                    pl.when(k + 1 < nb)(
                        lambda: issue(bstart + EB, 1 - b))
                    wait_gather(b)
                    pl.when(k >= 2)(lambda: wait_scatter(b))
                    compute(bstart, b, lo, hi, pq)
                    pltpu.async_copy(wsts[b], accum.at[idxs[b]],
                                     semss[b], add=True)
            return 0
        with jax.named_scope("edges"):
            pl.when(nb > 0)(lambda: issue(b0, 0))
            lax.fori_loop(0, nbp, pair_body, 0)
            pl.when(nb > 0)(lambda: wait_scatter(0))
            pl.when(nb > 1)(lambda: wait_scatter(1))
            plsc.subcore_barrier()
        with jax.named_scope("copyout"):
            pltpu.sync_copy(
                accum.at[pl.ds(s * TPS, TPS)],
                out.at[pl.ds(c * (NPASS * QR) + pq + s * TPS, TPS)])
            plsc.subcore_barrier()
        return 0
    lax.fori_loop(0, NPASS, pass_body, 0)


def _edge_stage_sc(ctab, qtab, edge_src, edge_dst):
    srcp = jnp.pad(edge_src, (0, GPAD - E), mode='edge')
    dstp = jnp.pad(edge_dst, (0, GPAD - E), mode='edge')
    mesh = plsc.VectorSubcoreMesh(core_axis_name="c", subcore_axis_name="s")
    f = functools.partial(
        pl.kernel,
        mesh=mesh,
        out_type=jax.ShapeDtypeStruct((2 * NPASS * QR, ROWW), jnp.float32),
        compiler_params=pltpu.CompilerParams(needs_layout_passes=False),
        scratch_types=[
            pltpu.VMEM((EPAD,), jnp.int32),
            pltpu.VMEM((EPAD,), jnp.int32),
            pltpu.VMEM((EB, 128), jnp.float32),
            pltpu.VMEM((EB, 128), jnp.float32),
            pltpu.VMEM((EB, 128), jnp.float32),
            pltpu.VMEM((EB, 128), jnp.float32),
            pltpu.VMEM((EB, ROWW), jnp.float32),
            pltpu.VMEM((EB, ROWW), jnp.float32),
            pltpu.VMEM((EB,), jnp.int32),
            pltpu.VMEM((EB,), jnp.int32),
            pltpu.VMEM((ZR, ROWW), jnp.float32),
            pltpu.VMEM_SHARED((ACC_ROWS, ROWW), jnp.float32),
            pltpu.SemaphoreType.DMA,
            pltpu.SemaphoreType.DMA,
            pltpu.SemaphoreType.DMA,
            pltpu.SemaphoreType.DMA,
        ])(_edge_body)
    res = f(ctab, qtab, srcp, dstp)
    return res[:N_F], res[NPASS * QR:NPASS * QR + N_F]


def kernel(truth_features, truth_features_0, fastsim_features, fastsim_global,
           edge_src, edge_dst,
           kW0, kb0, kW1, kb1, kW2, kb2, qW0, qb0, qW1, qb1, qW2, qb2,
           vW0, vb0, vW1, vb1, vW2, vb2, gWih, gWhh, gbih, gbhh,
           ln_g, ln_b, mW0, mb0, mW1, mb1):
    ws = dict(kW0=kW0, kb0=kb0, kW1=kW1, kb1=kb1, kW2=kW2, kb2=kb2,
              qW0=qW0, qb0=qb0, qW1=qW1, qb1=qb1, qW2=qW2, qb2=qb2,
              vW0=vW0, vb0=vb0, vW1=vW1, vb1=vb1, vW2=vW2, vb2=vb2,
              gWih=gWih, gWhh=gWhh, gbih=gbih, gbhh=gbhh,
              ln_g=ln_g, ln_b=ln_b, mW0=mW0, mb0=mb0, mW1=mW1, mb1=mb1)
    nodes_in = jnp.concatenate([truth_features, truth_features_0], axis=1)
    q_in = jnp.concatenate([fastsim_features, fastsim_global], axis=1)
    ctab, qtab = _pre_stage(nodes_in, q_in, truth_features_0, ws)
    pa, pb = _edge_stage_sc(ctab, qtab, edge_src, edge_dst)
    return _post_stage(pa, pb, fastsim_features, ws)


# speedup vs baseline: 1.3031x; 1.3031x over previous
import functools

import jax
import jax.numpy as jnp
from jax import lax
from jax.experimental import pallas as pl
from jax.experimental.pallas import tpu as pltpu
from jax.experimental.pallas import tpu_sc as plsc

N_T = 50000
N_F = 50000
E = 800000
D_T = 64
D_F = 64
D_SKIP = 32

BLK = 1000


def _pre_body(nodes_ref, q_in_ref, t0_ref,
              kW0, kb0, kW1, kb1, kW2, kb2,
              vW0, vb0, vW1, vb1, vW2, vb2,
              qW0, qb0, qW1, qb1, qW2, qb2,
              ctab_ref, qtab_ref):
    x = nodes_ref[...]
    h = jax.nn.relu(jnp.dot(x, kW0[...].T, preferred_element_type=jnp.float32) + kb0[...])
    h = jax.nn.relu(jnp.dot(h, kW1[...].T, preferred_element_type=jnp.float32) + kb1[...])
    k = jnp.dot(h, kW2[...].T, preferred_element_type=jnp.float32) + kb2[...]
    h = jax.nn.relu(jnp.dot(x, vW0[...].T, preferred_element_type=jnp.float32) + vb0[...])
    h = jax.nn.relu(jnp.dot(h, vW1[...].T, preferred_element_type=jnp.float32) + vb1[...])
    v = jnp.dot(h, vW2[...].T, preferred_element_type=jnp.float32) + vb2[...]
    xq = q_in_ref[...]
    h = jax.nn.relu(jnp.dot(xq, qW0[...].T, preferred_element_type=jnp.float32) + qb0[...])
    h = jax.nn.relu(jnp.dot(h, qW1[...].T, preferred_element_type=jnp.float32) + qb1[...])
    q = jnp.dot(h, qW2[...].T, preferred_element_type=jnp.float32) + qb2[...]
    z2 = jnp.zeros((x.shape[0], 2), dtype=jnp.float32)
    ctab_ref[...] = jnp.concatenate([k, z2, v, t0_ref[...]], axis=1)
    norm = jnp.float32(1.0) / jnp.sqrt(jnp.float32(30.0))
    qtab_ref[...] = jnp.concatenate([q * norm, z2], axis=1)


def _whole(arr2d):
    return pl.BlockSpec(arr2d, lambda i: (0, 0))


def _pre_stage(nodes_in, q_in, t0, ws):
    row = pl.BlockSpec((BLK, None), lambda i: (i, 0))

    def wspec(a):
        return pl.BlockSpec((a.shape[0], a.shape[1]), lambda i: (0, 0))

    def bspec(a):
        return pl.BlockSpec((1, a.shape[1]), lambda i: (0, 0))

    weights = [ws['kW0'], ws['kb0'], ws['kW1'], ws['kb1'], ws['kW2'], ws['kb2'],
               ws['vW0'], ws['vb0'], ws['vW1'], ws['vb1'], ws['vW2'], ws['vb2'],
               ws['qW0'], ws['qb0'], ws['qW1'], ws['qb1'], ws['qW2'], ws['qb2']]
    in_specs = [pl.BlockSpec((BLK, nodes_in.shape[1]), lambda i: (i, 0)),
                pl.BlockSpec((BLK, q_in.shape[1]), lambda i: (i, 0)),
                pl.BlockSpec((BLK, t0.shape[1]), lambda i: (i, 0))]
    for w in weights:
        if w.ndim == 1:
            in_specs.append(bspec(w.reshape(1, -1)))
        else:
            in_specs.append(wspec(w))
    weights_r = [w.reshape(1, -1) if w.ndim == 1 else w for w in weights]
    grid = N_T // BLK
    ctab, qtab = pl.pallas_call(
        _pre_body,
        grid=(grid,),
        in_specs=in_specs,
        out_specs=[pl.BlockSpec((BLK, 128), lambda i: (i, 0)),
                   pl.BlockSpec((BLK, 32), lambda i: (i, 0))],
        out_shape=[jax.ShapeDtypeStruct((N_T, 128), jnp.float32),
                   jax.ShapeDtypeStruct((N_F, 32), jnp.float32)],
    )(nodes_in, q_in, t0, *weights_r)
    return ctab, qtab


def _post_body(pa_ref, pb_ref, fs_ref,
               gWih, gWhh, gbih, gbhh, ln_g, ln_b, mW0, mb0, mW1, mb1,
               out_ref):
    pa = pa_ref[...]
    pb = pb_ref[...]
    num = pa[:, :96] + pb[:, :96]
    den = pa[:, 96:97] + pb[:, 96:97]
    den = jnp.where(den == 0.0, 1.0, den)
    ws = num / den
    fs = fs_ref[...]
    gi = jnp.dot(ws, gWih[...].T, preferred_element_type=jnp.float32) + gbih[...]
    gh = jnp.dot(fs, gWhh[...].T, preferred_element_type=jnp.float32) + gbhh[...]
    i_r, i_z, i_n = gi[:, :64], gi[:, 64:128], gi[:, 128:]
    h_r, h_z, h_n = gh[:, :64], gh[:, 64:128], gh[:, 128:]
    r = jax.nn.sigmoid(i_r + h_r)
    z = jax.nn.sigmoid(i_z + h_z)
    nn = jnp.tanh(i_n + r * h_n)
    h = (1.0 - z) * nn + z * fs
    mu = jnp.mean(h, axis=1, keepdims=True)
    var = jnp.mean((h - mu) ** 2, axis=1, keepdims=True)
    hn = (h - mu) * jax.lax.rsqrt(var + 1e-05) * ln_g[...] + ln_b[...]
    o = jax.nn.relu(jnp.dot(hn, mW0[...].T, preferred_element_type=jnp.float32) + mb0[...])
    o = jnp.dot(o, mW1[...].T, preferred_element_type=jnp.float32) + mb1[...]
    out_ref[...] = fs + o


def _post_stage(pa, pb, fs, ws):
    def wspec(a):
        return pl.BlockSpec((a.shape[0], a.shape[1]), lambda i: (0, 0))

    weights = [ws['gWih'], ws['gWhh'], ws['gbih'], ws['gbhh'], ws['ln_g'],
               ws['ln_b'], ws['mW0'], ws['mb0'], ws['mW1'], ws['mb1']]
    weights_r = [w.reshape(1, -1) if w.ndim == 1 else w for w in weights]
    in_specs = [pl.BlockSpec((BLK, pa.shape[1]), lambda i: (i, 0)),
                pl.BlockSpec((BLK, pb.shape[1]), lambda i: (i, 0)),
                pl.BlockSpec((BLK, fs.shape[1]), lambda i: (i, 0))]
    in_specs += [wspec(w) for w in weights_r]
    out = pl.pallas_call(
        _post_body,
        grid=(N_F // BLK,),
        in_specs=in_specs,
        out_specs=pl.BlockSpec((BLK, D_F), lambda i: (i, 0)),
        out_shape=jax.ShapeDtypeStruct((N_F, D_F), jnp.float32),
    )(pa, pb, fs, *weights_r)
    return out


NC = 2
NS = 16
L = 16
NW = NC * NS
EC = E // NW
EB = 64
NPASS = 10
QR = 5120
TPS = QR // NS
ROWW = 128
ACC_ROWS = QR + 8
DUMP = QR
ZR = 16
EPAD = 25088
NSCAN = EPAD // L
GPAD = 31 * EC + EPAD


def _edge_body(ctab, qtab, srcg, dstg, out,
               srcv, dstv, comb0, comb1, qb0, qb1,
               wst0, wst1, idx0, idx1, zbuf,
               accum, semg0, semg1, sems0, sems1):
    c = lax.axis_index("c")
    s = lax.axis_index("s")
    w = s * NC + c
    base_w = w * EC
    iv = lax.iota(jnp.int32, L)
    zv = jnp.zeros((L,), jnp.float32)

    combs = (comb0, comb1)
    qbufs = (qb0, qb1)
    wsts = (wst0, wst1)
    idxs = (idx0, idx1)
    semgs = (semg0, semg1)
    semss = (sems0, sems1)

    def zb_body(r, _):
        rr = jnp.broadcast_to(r, (L,)).astype(jnp.int32)
        for j in range(ROWW // L):
            plsc.store_scatter(zbuf, [rr, iv + j * L], zv)
        return 0
    lax.fori_loop(0, ZR, zb_body, 0)

    with jax.named_scope("idx_preload"):
        pltpu.sync_copy(srcg.at[pl.ds(base_w, EPAD)], srcv)
        pltpu.sync_copy(dstg.at[pl.ds(base_w, EPAD)], dstv)

    def cbody(g, cs):
        d = dstv[pl.ds(g * L, L)]
        gidx = g * L + iv
        d = jnp.where(gidx < EC, d, jnp.int32(2 ** 30))
        return tuple(
            cs[t] + jnp.sum((d < (t + 1) * QR).astype(jnp.int32))
            for t in range(NPASS - 1))
    with jax.named_scope("count_scan"):
        cs = lax.fori_loop(0, NSCAN, cbody,
                           tuple(jnp.int32(0) for _ in range(NPASS - 1)))

    NSPL = 4

    def issue(bstart, par):
        sw = EB // NSPL
        for t in range(NSPL):
            pltpu.async_copy(
                ctab.at[srcv.at[pl.ds(bstart + t * sw, sw)]],
                combs[par].at[pl.ds(t * sw, sw)], semgs[par])
        for t in range(2):
            pltpu.async_copy(
                qtab.at[dstv.at[pl.ds(bstart + t * (EB // 2), EB // 2)]],
                qbufs[par].at[pl.ds(t * (EB // 2), EB // 2)], semgs[par])

    def wait_gather(par):
        pltpu.make_async_copy(ctab.at[pl.ds(0, EB)], combs[par],
                              semgs[par]).wait()
        pltpu.make_async_copy(qtab.at[pl.ds(0, EB)], qbufs[par],
                              semgs[par]).wait()

    def wait_scatter(par):
        pltpu.make_async_copy(out.at[pl.ds(0, EB)],
                              accum.at[pl.ds(0, EB)], semss[par]).wait()

    def compute(bstart, par, lo, hi, pq):
        cb, qb, wb, ib = combs[par], qbufs[par], wsts[par], idxs[par]

        def gbody(g, _):
            rows16 = iv + g * L
            gidx = bstart + rows16
            d16 = dstv[pl.ds(bstart + g * L, L)]
            valid = (gidx >= lo) & (gidx < hi)
            dl = jnp.where(valid, d16 - pq, jnp.int32(DUMP))
            plsc.store_scatter(ib, [rows16], dl)
            att = jnp.zeros((L,), jnp.float32)
            for f in range(32):
                fc = jnp.full((L,), f, jnp.int32)
                att = att + (plsc.load_gather(cb, [rows16, fc]) *
                             plsc.load_gather(qb, [rows16, fc]))
            ex16 = jnp.exp(att)
            plsc.store_scatter(wb, [rows16, jnp.full((L,), 96, jnp.int32)],
                               ex16)
            for e in range(L):
                row = g * L + e
                exe = ex16[e]
                for j in range(6):
                    vv = cb[row, pl.ds(32 + j * L, L)]
                    wb[row, pl.ds(j * L, L)] = exe * vv
            return 0
        lax.fori_loop(0, EB // L, gbody, 0)

    def pass_body(p, _):
        lo = jnp.int32(0)
        for t in range(NPASS - 1):
            lo = jnp.where(p == t + 1, cs[t], lo)
        hi = jnp.int32(EC)
        for t in range(NPASS - 1):
            hi = jnp.where(p == t, cs[t], hi)
        pq = p * QR
        def zr_body(i, _):
            pltpu.sync_copy(zbuf, accum.at[pl.ds(s * TPS + i * ZR, ZR)])
            return 0
        with jax.named_scope("zero"):
            lax.fori_loop(0, TPS // ZR, zr_body, 0)
            plsc.subcore_barrier()

        b0 = (lo // EB) * EB
        nb = (hi - b0 + (EB - 1)) // EB
        nbp = (nb + 1) // 2

        def pair_body(i2, _):
            for b in range(2):
                k = i2 * 2 + b

                @pl.when(k < nb)
                def _():
                    bstart = b0 + k * EB
                    pl.when(k + 1 < nb)(
                        lambda: issue(bstart + EB, 1 - b))
                    wait_gather(b)
                    pl.when(k >= 2)(lambda: wait_scatter(b))
                    compute(bstart, b, lo, hi, pq)
                    pltpu.async_copy(wsts[b], accum.at[idxs[b]],
                                     semss[b], add=True)
            return 0
        with jax.named_scope("edges"):
            pl.when(nb > 0)(lambda: issue(b0, 0))
            lax.fori_loop(0, nbp, pair_body, 0)
            pl.when(nb > 0)(lambda: wait_scatter(0))
            pl.when(nb > 1)(lambda: wait_scatter(1))
            plsc.subcore_barrier()
        with jax.named_scope("copyout"):
            pltpu.sync_copy(
                accum.at[pl.ds(s * TPS, TPS)],
                out.at[pl.ds(c * (NPASS * QR) + pq + s * TPS, TPS)])
            plsc.subcore_barrier()
        return 0
    lax.fori_loop(0, NPASS, pass_body, 0)


def _edge_stage_sc(ctab, qtab, edge_src, edge_dst):
    srcp = jnp.pad(edge_src, (0, GPAD - E), mode='edge')
    dstp = jnp.pad(edge_dst, (0, GPAD - E), mode='edge')
    mesh = plsc.VectorSubcoreMesh(core_axis_name="c", subcore_axis_name="s")
    f = functools.partial(
        pl.kernel,
        mesh=mesh,
        out_type=jax.ShapeDtypeStruct((2 * NPASS * QR, ROWW), jnp.float32),
        compiler_params=pltpu.CompilerParams(
            needs_layout_passes=False, use_tc_tiling_on_sc=False),
        scratch_types=[
            pltpu.VMEM((EPAD,), jnp.int32),
            pltpu.VMEM((EPAD,), jnp.int32),
            pltpu.VMEM((EB, 128), jnp.float32),
            pltpu.VMEM((EB, 128), jnp.float32),
            pltpu.VMEM((EB, 32), jnp.float32),
            pltpu.VMEM((EB, 32), jnp.float32),
            pltpu.VMEM((EB, ROWW), jnp.float32),
            pltpu.VMEM((EB, ROWW), jnp.float32),
            pltpu.VMEM((EB,), jnp.int32),
            pltpu.VMEM((EB,), jnp.int32),
            pltpu.VMEM((ZR, ROWW), jnp.float32),
            pltpu.VMEM_SHARED((ACC_ROWS, ROWW), jnp.float32),
            pltpu.SemaphoreType.DMA,
            pltpu.SemaphoreType.DMA,
            pltpu.SemaphoreType.DMA,
            pltpu.SemaphoreType.DMA,
        ])(_edge_body)
    res = f(ctab, qtab, srcp, dstp)
    return res[:N_F], res[NPASS * QR:NPASS * QR + N_F]


def kernel(truth_features, truth_features_0, fastsim_features, fastsim_global,
           edge_src, edge_dst,
           kW0, kb0, kW1, kb1, kW2, kb2, qW0, qb0, qW1, qb1, qW2, qb2,
           vW0, vb0, vW1, vb1, vW2, vb2, gWih, gWhh, gbih, gbhh,
           ln_g, ln_b, mW0, mb0, mW1, mb1):
    ws = dict(kW0=kW0, kb0=kb0, kW1=kW1, kb1=kb1, kW2=kW2, kb2=kb2,
              qW0=qW0, qb0=qb0, qW1=qW1, qb1=qb1, qW2=qW2, qb2=qb2,
              vW0=vW0, vb0=vb0, vW1=vW1, vb1=vb1, vW2=vW2, vb2=vb2,
              gWih=gWih, gWhh=gWhh, gbih=gbih, gbhh=gbhh,
              ln_g=ln_g, ln_b=ln_b, mW0=mW0, mb0=mb0, mW1=mW1, mb1=mb1)
    nodes_in = jnp.concatenate([truth_features, truth_features_0], axis=1)
    q_in = jnp.concatenate([fastsim_features, fastsim_global], axis=1)
    ctab, qtab = _pre_stage(nodes_in, q_in, truth_features_0, ws)
    pa, pb = _edge_stage_sc(ctab, qtab, edge_src, edge_dst)
    return _post_stage(pa, pb, fastsim_features, ws)


# speedup vs baseline: 1.4545x; 1.1162x over previous
import functools

import jax
import jax.numpy as jnp
from jax import lax
from jax.experimental import pallas as pl
from jax.experimental.pallas import tpu as pltpu
from jax.experimental.pallas import tpu_sc as plsc

N_T = 50000
N_F = 50000
E = 800000
D_T = 64
D_F = 64
D_SKIP = 32

BLK = 1000


def _pre_body(nodes_ref, q_in_ref, t0_ref,
              kW0, kb0, kW1, kb1, kW2, kb2,
              vW0, vb0, vW1, vb1, vW2, vb2,
              qW0, qb0, qW1, qb1, qW2, qb2,
              ctab_ref, qtab_ref):
    x = nodes_ref[...]
    h = jax.nn.relu(jnp.dot(x, kW0[...].T, preferred_element_type=jnp.float32) + kb0[...])
    h = jax.nn.relu(jnp.dot(h, kW1[...].T, preferred_element_type=jnp.float32) + kb1[...])
    k = jnp.dot(h, kW2[...].T, preferred_element_type=jnp.float32) + kb2[...]
    h = jax.nn.relu(jnp.dot(x, vW0[...].T, preferred_element_type=jnp.float32) + vb0[...])
    h = jax.nn.relu(jnp.dot(h, vW1[...].T, preferred_element_type=jnp.float32) + vb1[...])
    v = jnp.dot(h, vW2[...].T, preferred_element_type=jnp.float32) + vb2[...]
    xq = q_in_ref[...]
    h = jax.nn.relu(jnp.dot(xq, qW0[...].T, preferred_element_type=jnp.float32) + qb0[...])
    h = jax.nn.relu(jnp.dot(h, qW1[...].T, preferred_element_type=jnp.float32) + qb1[...])
    q = jnp.dot(h, qW2[...].T, preferred_element_type=jnp.float32) + qb2[...]
    z2 = jnp.zeros((x.shape[0], 2), dtype=jnp.float32)
    ctab_ref[...] = jnp.concatenate([k, z2, v, t0_ref[...]], axis=1)
    norm = jnp.float32(1.0) / jnp.sqrt(jnp.float32(30.0))
    qtab_ref[...] = jnp.concatenate([q * norm, z2], axis=1)


def _whole(arr2d):
    return pl.BlockSpec(arr2d, lambda i: (0, 0))


def _pre_stage(nodes_in, q_in, t0, ws):
    row = pl.BlockSpec((BLK, None), lambda i: (i, 0))

    def wspec(a):
        return pl.BlockSpec((a.shape[0], a.shape[1]), lambda i: (0, 0))

    def bspec(a):
        return pl.BlockSpec((1, a.shape[1]), lambda i: (0, 0))

    weights = [ws['kW0'], ws['kb0'], ws['kW1'], ws['kb1'], ws['kW2'], ws['kb2'],
               ws['vW0'], ws['vb0'], ws['vW1'], ws['vb1'], ws['vW2'], ws['vb2'],
               ws['qW0'], ws['qb0'], ws['qW1'], ws['qb1'], ws['qW2'], ws['qb2']]
    in_specs = [pl.BlockSpec((BLK, nodes_in.shape[1]), lambda i: (i, 0)),
                pl.BlockSpec((BLK, q_in.shape[1]), lambda i: (i, 0)),
                pl.BlockSpec((BLK, t0.shape[1]), lambda i: (i, 0))]
    for w in weights:
        if w.ndim == 1:
            in_specs.append(bspec(w.reshape(1, -1)))
        else:
            in_specs.append(wspec(w))
    weights_r = [w.reshape(1, -1) if w.ndim == 1 else w for w in weights]
    grid = N_T // BLK
    ctab, qtab = pl.pallas_call(
        _pre_body,
        grid=(grid,),
        in_specs=in_specs,
        out_specs=[pl.BlockSpec((BLK, 128), lambda i: (i, 0)),
                   pl.BlockSpec((BLK, 32), lambda i: (i, 0))],
        out_shape=[jax.ShapeDtypeStruct((N_T, 128), jnp.float32),
                   jax.ShapeDtypeStruct((N_F, 32), jnp.float32)],
    )(nodes_in, q_in, t0, *weights_r)
    return ctab, qtab


def _post_body(pa_ref, pb_ref, fs_ref,
               gWih, gWhh, gbih, gbhh, ln_g, ln_b, mW0, mb0, mW1, mb1,
               out_ref):
    pa = pa_ref[...]
    pb = pb_ref[...]
    num = pa[:, :96] + pb[:, :96]
    den = pa[:, 96:97] + pb[:, 96:97]
    den = jnp.where(den == 0.0, 1.0, den)
    ws = num / den
    fs = fs_ref[...]
    gi = jnp.dot(ws, gWih[...].T, preferred_element_type=jnp.float32) + gbih[...]
    gh = jnp.dot(fs, gWhh[...].T, preferred_element_type=jnp.float32) + gbhh[...]
    i_r, i_z, i_n = gi[:, :64], gi[:, 64:128], gi[:, 128:]
    h_r, h_z, h_n = gh[:, :64], gh[:, 64:128], gh[:, 128:]
    r = jax.nn.sigmoid(i_r + h_r)
    z = jax.nn.sigmoid(i_z + h_z)
    nn = jnp.tanh(i_n + r * h_n)
    h = (1.0 - z) * nn + z * fs
    mu = jnp.mean(h, axis=1, keepdims=True)
    var = jnp.mean((h - mu) ** 2, axis=1, keepdims=True)
    hn = (h - mu) * jax.lax.rsqrt(var + 1e-05) * ln_g[...] + ln_b[...]
    o = jax.nn.relu(jnp.dot(hn, mW0[...].T, preferred_element_type=jnp.float32) + mb0[...])
    o = jnp.dot(o, mW1[...].T, preferred_element_type=jnp.float32) + mb1[...]
    out_ref[...] = fs + o


def _post_stage(pa, pb, fs, ws):
    def wspec(a):
        return pl.BlockSpec((a.shape[0], a.shape[1]), lambda i: (0, 0))

    weights = [ws['gWih'], ws['gWhh'], ws['gbih'], ws['gbhh'], ws['ln_g'],
               ws['ln_b'], ws['mW0'], ws['mb0'], ws['mW1'], ws['mb1']]
    weights_r = [w.reshape(1, -1) if w.ndim == 1 else w for w in weights]
    in_specs = [pl.BlockSpec((BLK, pa.shape[1]), lambda i: (i, 0)),
                pl.BlockSpec((BLK, pb.shape[1]), lambda i: (i, 0)),
                pl.BlockSpec((BLK, fs.shape[1]), lambda i: (i, 0))]
    in_specs += [wspec(w) for w in weights_r]
    out = pl.pallas_call(
        _post_body,
        grid=(N_F // BLK,),
        in_specs=in_specs,
        out_specs=pl.BlockSpec((BLK, D_F), lambda i: (i, 0)),
        out_shape=jax.ShapeDtypeStruct((N_F, D_F), jnp.float32),
    )(pa, pb, fs, *weights_r)
    return out


NC = 2
NS = 16
L = 16
NW = NC * NS
EC = E // NW
EB = 128
NPASS = 8
QR = 6656
TPS = QR // NS
ROWW = 128
ACC_ROWS = QR + 8
DUMP = QR
ZR = 16
SCHUNK = 1024
NSC = 25
EPAD = NSC * SCHUNK
GPAD = 31 * EC + EPAD


def _edge_body(ctab, qtab, srcg, dstg, out,
               sbuf, srcb0, srcb1, dstb0, dstb1, comb0, comb1, qb0, qb1,
               wst0, wst1, idx0, idx1, zbuf,
               accum, semg0, semg1, sems0, sems1):
    c = lax.axis_index("c")
    s = lax.axis_index("s")
    w = s * NC + c
    base_w = w * EC
    iv = lax.iota(jnp.int32, L)
    zv = jnp.zeros((L,), jnp.float32)

    srcbs = (srcb0, srcb1)
    dstbs = (dstb0, dstb1)
    combs = (comb0, comb1)
    qbufs = (qb0, qb1)
    wsts = (wst0, wst1)
    idxs = (idx0, idx1)
    semgs = (semg0, semg1)
    semss = (sems0, sems1)

    def zb_body(r, _):
        rr = jnp.broadcast_to(r, (L,)).astype(jnp.int32)
        for j in range(ROWW // L):
            plsc.store_scatter(zbuf, [rr, iv + j * L], zv)
        return 0
    lax.fori_loop(0, ZR, zb_body, 0)

    def cchunk(i, carry):
        pltpu.sync_copy(dstg.at[pl.ds(base_w + i * SCHUNK, SCHUNK)], sbuf)

        def cbody(g, cs):
            d = sbuf[pl.ds(g * L, L)]
            gidx = i * SCHUNK + g * L + iv
            d = jnp.where(gidx < EC, d, jnp.int32(2 ** 30))
            return tuple(
                cs[t] + jnp.sum((d < (t + 1) * QR).astype(jnp.int32))
                for t in range(NPASS - 1))
        return lax.fori_loop(0, SCHUNK // L, cbody, carry)
    with jax.named_scope("count_scan"):
        cs = lax.fori_loop(0, NSC, cchunk,
                           tuple(jnp.int32(0) for _ in range(NPASS - 1)))

    def issue(bstart, par):
        pltpu.sync_copy(srcg.at[pl.ds(base_w + bstart, EB)], srcbs[par])
        pltpu.sync_copy(dstg.at[pl.ds(base_w + bstart, EB)], dstbs[par])
        pltpu.async_copy(ctab.at[srcbs[par]], combs[par], semgs[par])
        pltpu.async_copy(qtab.at[dstbs[par]], qbufs[par], semgs[par])

    def wait_gather(par):
        pltpu.make_async_copy(ctab.at[pl.ds(0, EB)], combs[par],
                              semgs[par]).wait()
        pltpu.make_async_copy(qtab.at[pl.ds(0, EB)], qbufs[par],
                              semgs[par]).wait()

    def wait_scatter(par):
        pltpu.make_async_copy(out.at[pl.ds(0, EB)],
                              accum.at[pl.ds(0, EB)], semss[par]).wait()

    def compute(bstart, par, lo, hi, pq):
        cb, qb, wb, ib = combs[par], qbufs[par], wsts[par], idxs[par]
        db = dstbs[par]

        def gbody(g, _):
            rows16 = iv + g * L
            gidx = bstart + rows16
            d16 = db[pl.ds(g * L, L)]
            valid = (gidx >= lo) & (gidx < hi)
            dl = jnp.where(valid, d16 - pq, jnp.int32(DUMP))
            plsc.store_scatter(ib, [rows16], dl)
            att = jnp.zeros((L,), jnp.float32)
            for f in range(32):
                fc = jnp.full((L,), f, jnp.int32)
                att = att + (plsc.load_gather(cb, [rows16, fc]) *
                             plsc.load_gather(qb, [rows16, fc]))
            ex16 = jnp.exp(att)
            plsc.store_scatter(wb, [rows16, jnp.full((L,), 96, jnp.int32)],
                               ex16)
            for e in range(L):
                row = g * L + e
                exe = ex16[e]
                for j in range(6):
                    vv = cb[row, pl.ds(32 + j * L, L)]
                    wb[row, pl.ds(j * L, L)] = exe * vv
            return 0
        lax.fori_loop(0, EB // L, gbody, 0)

    def pass_body(p, _):
        lo = jnp.int32(0)
        for t in range(NPASS - 1):
            lo = jnp.where(p == t + 1, cs[t], lo)
        hi = jnp.int32(EC)
        for t in range(NPASS - 1):
            hi = jnp.where(p == t, cs[t], hi)
        pq = p * QR
        def zr_body(i, _):
            pltpu.sync_copy(zbuf, accum.at[pl.ds(s * TPS + i * ZR, ZR)])
            return 0
        with jax.named_scope("zero"):
            lax.fori_loop(0, TPS // ZR, zr_body, 0)
            plsc.subcore_barrier()

        b0 = (lo // EB) * EB
        nb = (hi - b0 + (EB - 1)) // EB
        nbp = (nb + 1) // 2

        def pair_body(i2, _):
            for b in range(2):
                k = i2 * 2 + b

                @pl.when(k < nb)
                def _():
                    bstart = b0 + k * EB
                    pl.when(k + 1 < nb)(
                        lambda: issue(bstart + EB, 1 - b))
                    wait_gather(b)
                    pl.when(k >= 2)(lambda: wait_scatter(b))
                    compute(bstart, b, lo, hi, pq)
                    pltpu.async_copy(wsts[b], accum.at[idxs[b]],
                                     semss[b], add=True)
            return 0
        with jax.named_scope("edges"):
            pl.when(nb > 0)(lambda: issue(b0, 0))
            lax.fori_loop(0, nbp, pair_body, 0)
            pl.when(nb > 0)(lambda: wait_scatter(0))
            pl.when(nb > 1)(lambda: wait_scatter(1))
            plsc.subcore_barrier()
        with jax.named_scope("copyout"):
            pltpu.sync_copy(
                accum.at[pl.ds(s * TPS, TPS)],
                out.at[pl.ds(c * (NPASS * QR) + pq + s * TPS, TPS)])
            plsc.subcore_barrier()
        return 0
    lax.fori_loop(0, NPASS, pass_body, 0)


def _edge_stage_sc(ctab, qtab, edge_src, edge_dst):
    srcp = jnp.pad(edge_src, (0, GPAD - E), mode='edge')
    dstp = jnp.pad(edge_dst, (0, GPAD - E), mode='edge')
    mesh = plsc.VectorSubcoreMesh(core_axis_name="c", subcore_axis_name="s")
    f = functools.partial(
        pl.kernel,
        mesh=mesh,
        out_type=jax.ShapeDtypeStruct((2 * NPASS * QR, ROWW), jnp.float32),
        compiler_params=pltpu.CompilerParams(
            needs_layout_passes=False, use_tc_tiling_on_sc=False),
        scratch_types=[
            pltpu.VMEM((SCHUNK,), jnp.int32),
            pltpu.VMEM((EB,), jnp.int32),
            pltpu.VMEM((EB,), jnp.int32),
            pltpu.VMEM((EB,), jnp.int32),
            pltpu.VMEM((EB,), jnp.int32),
            pltpu.VMEM((EB, 128), jnp.float32),
            pltpu.VMEM((EB, 128), jnp.float32),
            pltpu.VMEM((EB, 32), jnp.float32),
            pltpu.VMEM((EB, 32), jnp.float32),
            pltpu.VMEM((EB, ROWW), jnp.float32),
            pltpu.VMEM((EB, ROWW), jnp.float32),
            pltpu.VMEM((EB,), jnp.int32),
            pltpu.VMEM((EB,), jnp.int32),
            pltpu.VMEM((ZR, ROWW), jnp.float32),
            pltpu.VMEM_SHARED((ACC_ROWS, ROWW), jnp.float32),
            pltpu.SemaphoreType.DMA,
            pltpu.SemaphoreType.DMA,
            pltpu.SemaphoreType.DMA,
            pltpu.SemaphoreType.DMA,
        ])(_edge_body)
    res = f(ctab, qtab, srcp, dstp)
    return res[:N_F], res[NPASS * QR:NPASS * QR + N_F]


def kernel(truth_features, truth_features_0, fastsim_features, fastsim_global,
           edge_src, edge_dst,
           kW0, kb0, kW1, kb1, kW2, kb2, qW0, qb0, qW1, qb1, qW2, qb2,
           vW0, vb0, vW1, vb1, vW2, vb2, gWih, gWhh, gbih, gbhh,
           ln_g, ln_b, mW0, mb0, mW1, mb1):
    ws = dict(kW0=kW0, kb0=kb0, kW1=kW1, kb1=kb1, kW2=kW2, kb2=kb2,
              qW0=qW0, qb0=qb0, qW1=qW1, qb1=qb1, qW2=qW2, qb2=qb2,
              vW0=vW0, vb0=vb0, vW1=vW1, vb1=vb1, vW2=vW2, vb2=vb2,
              gWih=gWih, gWhh=gWhh, gbih=gbih, gbhh=gbhh,
              ln_g=ln_g, ln_b=ln_b, mW0=mW0, mb0=mb0, mW1=mW1, mb1=mb1)
    nodes_in = jnp.concatenate([truth_features, truth_features_0], axis=1)
    q_in = jnp.concatenate([fastsim_features, fastsim_global], axis=1)
    ctab, qtab = _pre_stage(nodes_in, q_in, truth_features_0, ws)
    pa, pb = _edge_stage_sc(ctab, qtab, edge_src, edge_dst)
    return _post_stage(pa, pb, fastsim_features, ws)


# speedup vs baseline: 2.8584x; 1.9652x over previous
import functools

import jax
import jax.numpy as jnp
from jax import lax
from jax.experimental import pallas as pl
from jax.experimental.pallas import tpu as pltpu
from jax.experimental.pallas import tpu_sc as plsc

N_T = 50000
N_F = 50000
E = 800000
D_T = 64
D_F = 64
D_SKIP = 32

BLK = 1000


def _pre_body(nodes_ref, q_in_ref, t0_ref,
              kW0, kb0, kW1, kb1, kW2, kb2,
              vW0, vb0, vW1, vb1, vW2, vb2,
              qW0, qb0, qW1, qb1, qW2, qb2,
              ctab_ref, qtab_ref):
    x = nodes_ref[...]
    h = jax.nn.relu(jnp.dot(x, kW0[...].T, preferred_element_type=jnp.float32) + kb0[...])
    h = jax.nn.relu(jnp.dot(h, kW1[...].T, preferred_element_type=jnp.float32) + kb1[...])
    k = jnp.dot(h, kW2[...].T, preferred_element_type=jnp.float32) + kb2[...]
    h = jax.nn.relu(jnp.dot(x, vW0[...].T, preferred_element_type=jnp.float32) + vb0[...])
    h = jax.nn.relu(jnp.dot(h, vW1[...].T, preferred_element_type=jnp.float32) + vb1[...])
    v = jnp.dot(h, vW2[...].T, preferred_element_type=jnp.float32) + vb2[...]
    xq = q_in_ref[...]
    h = jax.nn.relu(jnp.dot(xq, qW0[...].T, preferred_element_type=jnp.float32) + qb0[...])
    h = jax.nn.relu(jnp.dot(h, qW1[...].T, preferred_element_type=jnp.float32) + qb1[...])
    q = jnp.dot(h, qW2[...].T, preferred_element_type=jnp.float32) + qb2[...]
    z2 = jnp.zeros((x.shape[0], 2), dtype=jnp.float32)
    ctab_ref[...] = jnp.concatenate([k, z2, v, t0_ref[...]], axis=1)
    norm = jnp.float32(1.0) / jnp.sqrt(jnp.float32(30.0))
    qtab_ref[...] = jnp.concatenate([q * norm, z2], axis=1)


def _whole(arr2d):
    return pl.BlockSpec(arr2d, lambda i: (0, 0))


def _pre_stage(nodes_in, q_in, t0, ws):
    row = pl.BlockSpec((BLK, None), lambda i: (i, 0))

    def wspec(a):
        return pl.BlockSpec((a.shape[0], a.shape[1]), lambda i: (0, 0))

    def bspec(a):
        return pl.BlockSpec((1, a.shape[1]), lambda i: (0, 0))

    weights = [ws['kW0'], ws['kb0'], ws['kW1'], ws['kb1'], ws['kW2'], ws['kb2'],
               ws['vW0'], ws['vb0'], ws['vW1'], ws['vb1'], ws['vW2'], ws['vb2'],
               ws['qW0'], ws['qb0'], ws['qW1'], ws['qb1'], ws['qW2'], ws['qb2']]
    in_specs = [pl.BlockSpec((BLK, nodes_in.shape[1]), lambda i: (i, 0)),
                pl.BlockSpec((BLK, q_in.shape[1]), lambda i: (i, 0)),
                pl.BlockSpec((BLK, t0.shape[1]), lambda i: (i, 0))]
    for w in weights:
        if w.ndim == 1:
            in_specs.append(bspec(w.reshape(1, -1)))
        else:
            in_specs.append(wspec(w))
    weights_r = [w.reshape(1, -1) if w.ndim == 1 else w for w in weights]
    grid = N_T // BLK
    ctab, qtab = pl.pallas_call(
        _pre_body,
        grid=(grid,),
        in_specs=in_specs,
        out_specs=[pl.BlockSpec((BLK, 128), lambda i: (i, 0)),
                   pl.BlockSpec((BLK, 32), lambda i: (i, 0))],
        out_shape=[jax.ShapeDtypeStruct((N_T, 128), jnp.float32),
                   jax.ShapeDtypeStruct((N_F, 32), jnp.float32)],
    )(nodes_in, q_in, t0, *weights_r)
    return ctab, qtab


def _post_body(pa_ref, pb_ref, fs_ref,
               gWih, gWhh, gbih, gbhh, ln_g, ln_b, mW0, mb0, mW1, mb1,
               out_ref):
    pa = pa_ref[...]
    pb = pb_ref[...]
    num = pa[:, :96] + pb[:, :96]
    den = pa[:, 96:97] + pb[:, 96:97]
    den = jnp.where(den == 0.0, 1.0, den)
    ws = num / den
    fs = fs_ref[...]
    gi = jnp.dot(ws, gWih[...].T, preferred_element_type=jnp.float32) + gbih[...]
    gh = jnp.dot(fs, gWhh[...].T, preferred_element_type=jnp.float32) + gbhh[...]
    i_r, i_z, i_n = gi[:, :64], gi[:, 64:128], gi[:, 128:]
    h_r, h_z, h_n = gh[:, :64], gh[:, 64:128], gh[:, 128:]
    r = jax.nn.sigmoid(i_r + h_r)
    z = jax.nn.sigmoid(i_z + h_z)
    nn = jnp.tanh(i_n + r * h_n)
    h = (1.0 - z) * nn + z * fs
    mu = jnp.mean(h, axis=1, keepdims=True)
    var = jnp.mean((h - mu) ** 2, axis=1, keepdims=True)
    hn = (h - mu) * jax.lax.rsqrt(var + 1e-05) * ln_g[...] + ln_b[...]
    o = jax.nn.relu(jnp.dot(hn, mW0[...].T, preferred_element_type=jnp.float32) + mb0[...])
    o = jnp.dot(o, mW1[...].T, preferred_element_type=jnp.float32) + mb1[...]
    out_ref[...] = fs + o


def _post_stage(pa, pb, fs, ws):
    def wspec(a):
        return pl.BlockSpec((a.shape[0], a.shape[1]), lambda i: (0, 0))

    weights = [ws['gWih'], ws['gWhh'], ws['gbih'], ws['gbhh'], ws['ln_g'],
               ws['ln_b'], ws['mW0'], ws['mb0'], ws['mW1'], ws['mb1']]
    weights_r = [w.reshape(1, -1) if w.ndim == 1 else w for w in weights]
    in_specs = [pl.BlockSpec((BLK, pa.shape[1]), lambda i: (i, 0)),
                pl.BlockSpec((BLK, pb.shape[1]), lambda i: (i, 0)),
                pl.BlockSpec((BLK, fs.shape[1]), lambda i: (i, 0))]
    in_specs += [wspec(w) for w in weights_r]
    out = pl.pallas_call(
        _post_body,
        grid=(N_F // BLK,),
        in_specs=in_specs,
        out_specs=pl.BlockSpec((BLK, D_F), lambda i: (i, 0)),
        out_shape=jax.ShapeDtypeStruct((N_F, D_F), jnp.float32),
    )(pa, pb, fs, *weights_r)
    return out


NC = 2
NS = 16
L = 16
NW = NC * NS
EC = E // NW
EB = 128
NPASS = 8
QR = 6656
TPS = QR // NS
ROWW = 128
ACC_ROWS = QR + 8
DUMP = QR
ZR = 16
SCHUNK = 1024
NSC = 25
EPAD = NSC * SCHUNK
GPAD = 31 * EC + EPAD


def _edge_body(ctab, qtab, srcg, dstg, out,
               sbuf, srcb0, srcb1, dstb0, dstb1, comb0, comb1, qb0, qb1,
               wst0, wst1, idx0, idx1, zbuf,
               accum, semg0, semg1, sems0, sems1):
    c = lax.axis_index("c")
    s = lax.axis_index("s")
    w = s * NC + c
    base_w = w * EC
    iv = lax.iota(jnp.int32, L)
    e0mask = jnp.where(iv == 0, jnp.float32(1.0), jnp.float32(0.0))
    zv = jnp.zeros((L,), jnp.float32)

    srcbs = (srcb0, srcb1)
    dstbs = (dstb0, dstb1)
    combs = (comb0, comb1)
    qbufs = (qb0, qb1)
    wsts = (wst0, wst1)
    idxs = (idx0, idx1)
    semgs = (semg0, semg1)
    semss = (sems0, sems1)

    def zb_body(r, _):
        rr = jnp.broadcast_to(r, (L,)).astype(jnp.int32)
        for j in range(ROWW // L):
            plsc.store_scatter(zbuf, [rr, iv + j * L], zv)
        return 0
    lax.fori_loop(0, ZR, zb_body, 0)

    def cchunk(i, carry):
        pltpu.sync_copy(dstg.at[pl.ds(base_w + i * SCHUNK, SCHUNK)], sbuf)

        def cbody(g, cs):
            d = sbuf[pl.ds(g * L, L)]
            gidx = i * SCHUNK + g * L + iv
            d = jnp.where(gidx < EC, d, jnp.int32(2 ** 30))
            return tuple(
                cs[t] + jnp.sum((d < (t + 1) * QR).astype(jnp.int32))
                for t in range(NPASS - 1))
        return lax.fori_loop(0, SCHUNK // L, cbody, carry)
    with jax.named_scope("count_scan"):
        cs = lax.fori_loop(0, NSC, cchunk,
                           tuple(jnp.int32(0) for _ in range(NPASS - 1)))

    def issue(bstart, par):
        pltpu.sync_copy(srcg.at[pl.ds(base_w + bstart, EB)], srcbs[par])
        pltpu.sync_copy(dstg.at[pl.ds(base_w + bstart, EB)], dstbs[par])
        pltpu.async_copy(ctab.at[srcbs[par]], combs[par], semgs[par])
        pltpu.async_copy(qtab.at[dstbs[par]], qbufs[par], semgs[par])

    def wait_gather(par):
        pltpu.make_async_copy(ctab.at[pl.ds(0, EB)], combs[par],
                              semgs[par]).wait()
        pltpu.make_async_copy(qtab.at[pl.ds(0, EB)], qbufs[par],
                              semgs[par]).wait()

    def wait_scatter(par):
        pltpu.make_async_copy(out.at[pl.ds(0, EB)],
                              accum.at[pl.ds(0, EB)], semss[par]).wait()

    def compute(bstart, par, lo, hi, pq):
        cb, qb, wb, ib = combs[par], qbufs[par], wsts[par], idxs[par]
        db = dstbs[par]

        def gbody(g, _):
            rows16 = iv + g * L
            gidx = bstart + rows16
            d16 = db[pl.ds(g * L, L)]
            valid = (gidx >= lo) & (gidx < hi)
            dl = jnp.where(valid, d16 - pq, jnp.int32(DUMP))
            plsc.store_scatter(ib, [rows16], dl)
            for e in range(L):
                row = g * L + e
                a = cb[row, pl.ds(0, L)] * qb[row, pl.ds(0, L)]
                a = a + cb[row, pl.ds(L, L)] * qb[row, pl.ds(L, L)]
                att = jnp.sum(a)
                exv = jnp.exp(jnp.broadcast_to(att, (L,)))
                for j in range(6):
                    vv = cb[row, pl.ds(32 + j * L, L)]
                    wb[row, pl.ds(j * L, L)] = exv * vv
                wb[row, pl.ds(96, L)] = exv * e0mask
            return 0
        lax.fori_loop(0, EB // L, gbody, 0)

    def pass_body(p, _):
        lo = jnp.int32(0)
        for t in range(NPASS - 1):
            lo = jnp.where(p == t + 1, cs[t], lo)
        hi = jnp.int32(EC)
        for t in range(NPASS - 1):
            hi = jnp.where(p == t, cs[t], hi)
        pq = p * QR
        def zr_body(i, _):
            pltpu.sync_copy(zbuf, accum.at[pl.ds(s * TPS + i * ZR, ZR)])
            return 0
        with jax.named_scope("zero"):
            lax.fori_loop(0, TPS // ZR, zr_body, 0)
            plsc.subcore_barrier()

        b0 = (lo // EB) * EB
        nb = (hi - b0 + (EB - 1)) // EB
        nbp = (nb + 1) // 2

        def pair_body(i2, _):
            for b in range(2):
                k = i2 * 2 + b

                @pl.when(k < nb)
                def _():
                    bstart = b0 + k * EB
                    pl.when(k + 1 < nb)(
                        lambda: issue(bstart + EB, 1 - b))
                    wait_gather(b)
                    pl.when(k >= 2)(lambda: wait_scatter(b))
                    compute(bstart, b, lo, hi, pq)
                    pltpu.async_copy(wsts[b], accum.at[idxs[b]],
                                     semss[b], add=True)
            return 0
        with jax.named_scope("edges"):
            pl.when(nb > 0)(lambda: issue(b0, 0))
            lax.fori_loop(0, nbp, pair_body, 0)
            pl.when(nb > 0)(lambda: wait_scatter(0))
            pl.when(nb > 1)(lambda: wait_scatter(1))
            plsc.subcore_barrier()
        with jax.named_scope("copyout"):
            pltpu.sync_copy(
                accum.at[pl.ds(s * TPS, TPS)],
                out.at[pl.ds(c * (NPASS * QR) + pq + s * TPS, TPS)])
            plsc.subcore_barrier()
        return 0
    lax.fori_loop(0, NPASS, pass_body, 0)


def _edge_stage_sc(ctab, qtab, edge_src, edge_dst):
    srcp = jnp.pad(edge_src, (0, GPAD - E), mode='edge')
    dstp = jnp.pad(edge_dst, (0, GPAD - E), mode='edge')
    mesh = plsc.VectorSubcoreMesh(core_axis_name="c", subcore_axis_name="s")
    f = functools.partial(
        pl.kernel,
        mesh=mesh,
        out_type=jax.ShapeDtypeStruct((2 * NPASS * QR, ROWW), jnp.float32),
        compiler_params=pltpu.CompilerParams(
            needs_layout_passes=False, use_tc_tiling_on_sc=False),
        scratch_types=[
            pltpu.VMEM((SCHUNK,), jnp.int32),
            pltpu.VMEM((EB,), jnp.int32),
            pltpu.VMEM((EB,), jnp.int32),
            pltpu.VMEM((EB,), jnp.int32),
            pltpu.VMEM((EB,), jnp.int32),
            pltpu.VMEM((EB, 128), jnp.float32),
            pltpu.VMEM((EB, 128), jnp.float32),
            pltpu.VMEM((EB, 32), jnp.float32),
            pltpu.VMEM((EB, 32), jnp.float32),
            pltpu.VMEM((EB, ROWW), jnp.float32),
            pltpu.VMEM((EB, ROWW), jnp.float32),
            pltpu.VMEM((EB,), jnp.int32),
            pltpu.VMEM((EB,), jnp.int32),
            pltpu.VMEM((ZR, ROWW), jnp.float32),
            pltpu.VMEM_SHARED((ACC_ROWS, ROWW), jnp.float32),
            pltpu.SemaphoreType.DMA,
            pltpu.SemaphoreType.DMA,
            pltpu.SemaphoreType.DMA,
            pltpu.SemaphoreType.DMA,
        ])(_edge_body)
    res = f(ctab, qtab, srcp, dstp)
    return res[:N_F], res[NPASS * QR:NPASS * QR + N_F]


def kernel(truth_features, truth_features_0, fastsim_features, fastsim_global,
           edge_src, edge_dst,
           kW0, kb0, kW1, kb1, kW2, kb2, qW0, qb0, qW1, qb1, qW2, qb2,
           vW0, vb0, vW1, vb1, vW2, vb2, gWih, gWhh, gbih, gbhh,
           ln_g, ln_b, mW0, mb0, mW1, mb1):
    ws = dict(kW0=kW0, kb0=kb0, kW1=kW1, kb1=kb1, kW2=kW2, kb2=kb2,
              qW0=qW0, qb0=qb0, qW1=qW1, qb1=qb1, qW2=qW2, qb2=qb2,
              vW0=vW0, vb0=vb0, vW1=vW1, vb1=vb1, vW2=vW2, vb2=vb2,
              gWih=gWih, gWhh=gWhh, gbih=gbih, gbhh=gbhh,
              ln_g=ln_g, ln_b=ln_b, mW0=mW0, mb0=mb0, mW1=mW1, mb1=mb1)
    nodes_in = jnp.concatenate([truth_features, truth_features_0], axis=1)
    q_in = jnp.concatenate([fastsim_features, fastsim_global], axis=1)
    ctab, qtab = _pre_stage(nodes_in, q_in, truth_features_0, ws)
    pa, pb = _edge_stage_sc(ctab, qtab, edge_src, edge_dst)
    return _post_stage(pa, pb, fastsim_features, ws)
